# Initial kernel scaffold; baseline (speedup 1.0000x reference)
import jax
import jax.numpy as jnp
from jax.experimental import pallas as pl


def kernel(prediction, target):
    raise NotImplementedError("write your pallas kernel here")


# speedup vs baseline: 17.1137x; 17.1137x over previous
import functools

import jax
import jax.numpy as jnp
from jax.experimental import pallas as pl

_TRIM = 0.2
_ALPHA = 0.5
_SCALES = 4
_H = 512
_W = 512
_N = _H * _W
_TRIM_OFF = int((1 - _TRIM) * _N)


def _ukey(x):
    b = jax.lax.bitcast_convert_type(x, jnp.uint32)
    return jnp.where(b >= jnp.uint32(0x80000000), ~b, b ^ jnp.uint32(0x80000000))


def _key_to_f32(r):
    b = jnp.where(r >= jnp.uint32(0x80000000), r ^ jnp.uint32(0x80000000), ~r)
    return jax.lax.bitcast_convert_type(b, jnp.float32)


def _kth_smallest(keys, k):
    def body(i, r):
        shift = jnp.uint32(31) - i.astype(jnp.uint32)
        cand = r | (jnp.uint32(1) << shift)
        cnt_less = jnp.sum((keys < cand).astype(jnp.int32))
        return jnp.where(cnt_less <= k, cand, r)

    return jax.lax.fori_loop(0, 32, body, jnp.uint32(0))


def _grad_term(pn, tn, s):
    mg = (tn > 0.0).astype(jnp.float32)

    dx = jnp.abs(pn[:, s:] - pn[:, :-s])
    mx = mg[:, s:] * mg[:, :-s]
    if s > 1:
        ri = jax.lax.broadcasted_iota(jnp.int32, dx.shape, 0)
        ci = jax.lax.broadcasted_iota(jnp.int32, dx.shape, 1)
        sel = ((ri & (s - 1)) == 0) & ((ci & (s - 1)) == 0)
        lx = jnp.sum(jnp.where(sel, dx * mx, 0.0))
    else:
        lx = jnp.sum(dx * mx)

    dy = jnp.abs(pn[s:, :] - pn[:-s, :])
    my = mg[s:, :] * mg[:-s, :]
    if s > 1:
        ri = jax.lax.broadcasted_iota(jnp.int32, dy.shape, 0)
        ci = jax.lax.broadcasted_iota(jnp.int32, dy.shape, 1)
        sel = ((ri & (s - 1)) == 0) & ((ci & (s - 1)) == 0)
        ly = jnp.sum(jnp.where(sel, dy * my, 0.0))
    else:
        ly = jnp.sum(dy * my)

    if s > 1:
        ri = jax.lax.broadcasted_iota(jnp.int32, mg.shape, 0)
        ci = jax.lax.broadcasted_iota(jnp.int32, mg.shape, 1)
        sel = ((ri & (s - 1)) == 0) & ((ci & (s - 1)) == 0)
        norm = jnp.sum(jnp.where(sel, mg, 0.0))
    else:
        norm = jnp.sum(mg)

    return jnp.where(norm > 0.0, (lx + ly) / norm, 0.0)


def _loss_kernel(pred_ref, targ_ref, out_ref):
    p = pred_ref[0]
    t = targ_ref[0]
    mask = t > 0.0
    maskf = mask.astype(jnp.float32)
    cnt_f = jnp.sum(maskf)
    cnt = cnt_f.astype(jnp.int32)
    k = jnp.maximum((cnt - 1) // 2, 0)

    inf = jnp.float32(jnp.inf)

    pk = _ukey(jnp.where(mask, p, inf))
    shift_p = _key_to_f32(_kth_smallest(pk, k))
    scale_p = jnp.sum(jnp.abs(p - shift_p) * maskf) / cnt_f
    pn = (p - shift_p) / scale_p

    tk = _ukey(jnp.where(mask, t, inf))
    shift_t = _key_to_f32(_kth_smallest(tk, k))
    scale_t = jnp.sum(jnp.abs(t - shift_t) * maskf) / cnt_f
    tn = (t - shift_t) / scale_t

    mask2 = tn > 0.0
    cnt2_f = jnp.sum(mask2.astype(jnp.float32))
    cnt2 = cnt2_f.astype(jnp.int32)
    res = jnp.where(mask2, jnp.abs(pn - tn), 0.0)
    max_index = jnp.minimum((_N - cnt2) + _TRIM_OFF, _N - 1)
    gi = max_index // _W
    gj = max_index - gi * _W
    ri = jax.lax.broadcasted_iota(jnp.int32, (_H, _W), 0)
    ci = jax.lax.broadcasted_iota(jnp.int32, (_H, _W), 1)
    max_res = jnp.sum(jnp.where((ri == gi) & (ci == gj), res, 0.0))
    res = jnp.where(res > max_res, 0.0, res)
    mae = jnp.where(cnt2 > 0, jnp.sum(res) / (2.0 * cnt2_f), 0.0)

    g1 = _grad_term(pn, tn, 1)
    g2 = _grad_term(pn, tn, 2)
    g4 = _grad_term(pn, tn, 4)
    g8 = _grad_term(pn, tn, 8)

    lane = jax.lax.broadcasted_iota(jnp.int32, (1, 128), 1)
    vec = (
        jnp.where(lane == 0, mae, 0.0)
        + jnp.where(lane == 1, g1, 0.0)
        + jnp.where(lane == 2, g2, 0.0)
        + jnp.where(lane == 3, g4, 0.0)
        + jnp.where(lane == 4, g8, 0.0)
    )
    out_ref[0] = vec


@jax.jit
def kernel(prediction, target):
    b = prediction.shape[0]
    out = pl.pallas_call(
        _loss_kernel,
        grid=(b,),
        in_specs=[
            pl.BlockSpec((1, _H, _W), lambda i: (i, 0, 0)),
            pl.BlockSpec((1, _H, _W), lambda i: (i, 0, 0)),
        ],
        out_specs=pl.BlockSpec((1, 1, 128), lambda i: (i, 0, 0)),
        out_shape=jax.ShapeDtypeStruct((b, 1, 128), jnp.float32),
    )(prediction, target)
    per_row = out[:, 0, :]
    loss = jnp.mean(per_row[:, 0]) + _ALPHA * (
        jnp.mean(per_row[:, 1])
        + jnp.mean(per_row[:, 2])
        + jnp.mean(per_row[:, 3])
        + jnp.mean(per_row[:, 4])
    )
    return loss


# speedup vs baseline: 34.2233x; 1.9998x over previous
import functools

import jax
import jax.numpy as jnp
from jax.experimental import pallas as pl

_TRIM = 0.2
_ALPHA = 0.5
_SCALES = 4
_H = 512
_W = 512
_N = _H * _W
_TRIM_OFF = int((1 - _TRIM) * _N)


def _ukey(x):
    b = jax.lax.bitcast_convert_type(x, jnp.uint32)
    return jnp.where(b >= jnp.uint32(0x80000000), ~b, b ^ jnp.uint32(0x80000000))


def _key_to_f32(r):
    b = jnp.where(r >= jnp.uint32(0x80000000), r ^ jnp.uint32(0x80000000), ~r)
    return jax.lax.bitcast_convert_type(b, jnp.float32)


def _count16(plane, c):
    cb = (c ^ 0x8000).astype(jnp.int16)
    x = (plane < cb).astype(jnp.int16)
    rows = x.shape[0]
    while rows > 8:
        half = rows // 2
        x = x[:half] + x[half:]
        rows = half
    return jnp.sum(x.astype(jnp.int32))


def _bisect16(pa, pb, ka, kb):
    def body(i, carry):
        ra, rb = carry
        bit = jnp.int32(1) << (jnp.int32(15) - i)
        ca = ra | bit
        cb = rb | bit
        na = _count16(pa, ca)
        nb = _count16(pb, cb)
        return jnp.where(na <= ka, ca, ra), jnp.where(nb <= kb, cb, rb)

    return jax.lax.fori_loop(0, 16, body, (jnp.int32(0), jnp.int32(0)))


def _biased16(u):
    return (u ^ jnp.uint32(0x8000)).astype(jnp.int16)


def _two_medians(keys_a, keys_b, k):
    hi_a = _biased16(keys_a >> 16)
    hi_b = _biased16(keys_b >> 16)
    rha, rhb = _bisect16(hi_a, hi_b, k, k)
    below_a = _count16(hi_a, rha)
    below_b = _count16(hi_b, rhb)
    sent = jnp.int16(0x7FFF)
    lo_a = jnp.where(hi_a == (rha ^ 0x8000).astype(jnp.int16),
                     _biased16(keys_a & jnp.uint32(0xFFFF)), sent)
    lo_b = jnp.where(hi_b == (rhb ^ 0x8000).astype(jnp.int16),
                     _biased16(keys_b & jnp.uint32(0xFFFF)), sent)
    rla, rlb = _bisect16(lo_a, lo_b, k - below_a, k - below_b)
    key_a = (rha.astype(jnp.uint32) << 16) | rla.astype(jnp.uint32)
    key_b = (rhb.astype(jnp.uint32) << 16) | rlb.astype(jnp.uint32)
    return key_a, key_b


def _grad_term(pn, tn, s):
    mg = (tn > 0.0).astype(jnp.float32)

    dx = jnp.abs(pn[:, s:] - pn[:, :-s])
    mx = mg[:, s:] * mg[:, :-s]
    if s > 1:
        ri = jax.lax.broadcasted_iota(jnp.int32, dx.shape, 0)
        ci = jax.lax.broadcasted_iota(jnp.int32, dx.shape, 1)
        sel = ((ri & (s - 1)) == 0) & ((ci & (s - 1)) == 0)
        lx = jnp.sum(jnp.where(sel, dx * mx, 0.0))
    else:
        lx = jnp.sum(dx * mx)

    dy = jnp.abs(pn[s:, :] - pn[:-s, :])
    my = mg[s:, :] * mg[:-s, :]
    if s > 1:
        ri = jax.lax.broadcasted_iota(jnp.int32, dy.shape, 0)
        ci = jax.lax.broadcasted_iota(jnp.int32, dy.shape, 1)
        sel = ((ri & (s - 1)) == 0) & ((ci & (s - 1)) == 0)
        ly = jnp.sum(jnp.where(sel, dy * my, 0.0))
    else:
        ly = jnp.sum(dy * my)

    if s > 1:
        ri = jax.lax.broadcasted_iota(jnp.int32, mg.shape, 0)
        ci = jax.lax.broadcasted_iota(jnp.int32, mg.shape, 1)
        sel = ((ri & (s - 1)) == 0) & ((ci & (s - 1)) == 0)
        norm = jnp.sum(jnp.where(sel, mg, 0.0))
    else:
        norm = jnp.sum(mg)

    return jnp.where(norm > 0.0, (lx + ly) / norm, 0.0)


def _loss_kernel(pred_ref, targ_ref, out_ref):
    p = pred_ref[0]
    t = targ_ref[0]
    mask = t > 0.0
    maskf = mask.astype(jnp.float32)
    cnt_f = jnp.sum(maskf)
    cnt = cnt_f.astype(jnp.int32)
    k = jnp.maximum((cnt - 1) // 2, 0)

    inf = jnp.float32(jnp.inf)

    pk = _ukey(jnp.where(mask, p, inf))
    tk = _ukey(jnp.where(mask, t, inf))
    med_p, med_t = _two_medians(pk, tk, k)
    shift_p = _key_to_f32(med_p)
    shift_t = _key_to_f32(med_t)
    scale_p = jnp.sum(jnp.abs(p - shift_p) * maskf) / cnt_f
    pn = (p - shift_p) / scale_p
    scale_t = jnp.sum(jnp.abs(t - shift_t) * maskf) / cnt_f
    tn = (t - shift_t) / scale_t

    mask2 = tn > 0.0
    cnt2_f = jnp.sum(mask2.astype(jnp.float32))
    cnt2 = cnt2_f.astype(jnp.int32)
    res = jnp.where(mask2, jnp.abs(pn - tn), 0.0)
    max_index = jnp.minimum((_N - cnt2) + _TRIM_OFF, _N - 1)
    gi = max_index // _W
    gj = max_index - gi * _W
    ri = jax.lax.broadcasted_iota(jnp.int32, (_H, _W), 0)
    ci = jax.lax.broadcasted_iota(jnp.int32, (_H, _W), 1)
    max_res = jnp.sum(jnp.where((ri == gi) & (ci == gj), res, 0.0))
    res = jnp.where(res > max_res, 0.0, res)
    mae = jnp.where(cnt2 > 0, jnp.sum(res) / (2.0 * cnt2_f), 0.0)

    g1 = _grad_term(pn, tn, 1)
    g2 = _grad_term(pn, tn, 2)
    g4 = _grad_term(pn, tn, 4)
    g8 = _grad_term(pn, tn, 8)

    lane = jax.lax.broadcasted_iota(jnp.int32, (1, 128), 1)
    vec = (
        jnp.where(lane == 0, mae, 0.0)
        + jnp.where(lane == 1, g1, 0.0)
        + jnp.where(lane == 2, g2, 0.0)
        + jnp.where(lane == 3, g4, 0.0)
        + jnp.where(lane == 4, g8, 0.0)
    )
    out_ref[0] = vec


@jax.jit
def kernel(prediction, target):
    b = prediction.shape[0]
    out = pl.pallas_call(
        _loss_kernel,
        grid=(b,),
        in_specs=[
            pl.BlockSpec((1, _H, _W), lambda i: (i, 0, 0)),
            pl.BlockSpec((1, _H, _W), lambda i: (i, 0, 0)),
        ],
        out_specs=pl.BlockSpec((1, 1, 128), lambda i: (i, 0, 0)),
        out_shape=jax.ShapeDtypeStruct((b, 1, 128), jnp.float32),
    )(prediction, target)
    per_row = out[:, 0, :]
    loss = jnp.mean(per_row[:, 0]) + _ALPHA * (
        jnp.mean(per_row[:, 1])
        + jnp.mean(per_row[:, 2])
        + jnp.mean(per_row[:, 3])
        + jnp.mean(per_row[:, 4])
    )
    return loss


# speedup vs baseline: 40.9673x; 1.1971x over previous
import functools

import jax
import jax.numpy as jnp
from jax.experimental import pallas as pl
from jax.experimental.pallas import tpu as pltpu

_TRIM = 0.2
_ALPHA = 0.5
_SCALES = 4
_H = 512
_W = 512
_N = _H * _W
_TRIM_OFF = int((1 - _TRIM) * _N)


def _ukey(x):
    b = jax.lax.bitcast_convert_type(x, jnp.uint32)
    return jnp.where(b >= jnp.uint32(0x80000000), ~b, b ^ jnp.uint32(0x80000000))


def _key_to_f32(r):
    b = jnp.where(r >= jnp.uint32(0x80000000), r ^ jnp.uint32(0x80000000), ~r)
    return jax.lax.bitcast_convert_type(b, jnp.float32)


def _count16(plane, c):
    cb = (c ^ 0x8000).astype(jnp.int16)
    x = (plane < cb).astype(jnp.int16)
    rows = x.shape[0]
    while rows > 8:
        half = rows // 2
        x = x[:half] + x[half:]
        rows = half
    return jnp.sum(x.astype(jnp.int32))


def _bisect16(pa, pb, ka, kb):
    def body(i, carry):
        ra, rb = carry
        bit = jnp.int32(1) << (jnp.int32(15) - i)
        ca = ra | bit
        cb = rb | bit
        na = _count16(pa, ca)
        nb = _count16(pb, cb)
        return jnp.where(na <= ka, ca, ra), jnp.where(nb <= kb, cb, rb)

    return jax.lax.fori_loop(0, 16, body, (jnp.int32(0), jnp.int32(0)))


def _biased16(u):
    return (u ^ jnp.uint32(0x8000)).astype(jnp.int16)


def _two_medians(keys_a, keys_b, k):
    hi_a = _biased16(keys_a >> 16)
    hi_b = _biased16(keys_b >> 16)
    rha, rhb = _bisect16(hi_a, hi_b, k, k)
    below_a = _count16(hi_a, rha)
    below_b = _count16(hi_b, rhb)
    sent = jnp.int16(0x7FFF)
    lo_a = jnp.where(hi_a == (rha ^ 0x8000).astype(jnp.int16),
                     _biased16(keys_a & jnp.uint32(0xFFFF)), sent)
    lo_b = jnp.where(hi_b == (rhb ^ 0x8000).astype(jnp.int16),
                     _biased16(keys_b & jnp.uint32(0xFFFF)), sent)
    rla, rlb = _bisect16(lo_a, lo_b, k - below_a, k - below_b)
    key_a = (rha.astype(jnp.uint32) << 16) | rla.astype(jnp.uint32)
    key_b = (rhb.astype(jnp.uint32) << 16) | rlb.astype(jnp.uint32)
    return key_a, key_b


def _grad_term(pn_ref, tn_ref, s):
    sl = pl.Slice(0, _H // s, s)
    ps = jnp.concatenate([pn_ref[g, sl, :] for g in range(4)], axis=1)
    ts = jnp.concatenate([tn_ref[g, sl, :] for g in range(4)], axis=1)
    mg = (ts > 0.0).astype(jnp.float32)
    if s > 1:
        ci = jax.lax.broadcasted_iota(jnp.int32, mg.shape, 1)
        csel = (ci & (s - 1)) == 0
        mg = jnp.where(csel, mg, 0.0)
        dx = jnp.abs(ps[:, s:] - ps[:, :-s])
        lx = jnp.sum(dx * (mg[:, s:] * mg[:, :-s]))
        dy = jnp.abs(ps[1:, :] - ps[:-1, :])
        ly = jnp.sum(dy * (mg[1:, :] * mg[:-1, :]))
    else:
        dx = jnp.abs(ps[:, 1:] - ps[:, :-1])
        lx = jnp.sum(dx * (mg[:, 1:] * mg[:, :-1]))
        dy = jnp.abs(ps[1:, :] - ps[:-1, :])
        ly = jnp.sum(dy * (mg[1:, :] * mg[:-1, :]))
    norm = jnp.sum(mg)
    return jnp.where(norm > 0.0, (lx + ly) / norm, 0.0)


def _loss_kernel(pred_ref, targ_ref, out_ref, pn_ref, tn_ref):
    p = pred_ref[0]
    t = targ_ref[0]
    mask = t > 0.0
    maskf = mask.astype(jnp.float32)
    cnt_f = jnp.sum(maskf)
    cnt = cnt_f.astype(jnp.int32)
    k = jnp.maximum((cnt - 1) // 2, 0)

    inf = jnp.float32(jnp.inf)

    pk = _ukey(jnp.where(mask, p, inf))
    tk = _ukey(jnp.where(mask, t, inf))
    med_p, med_t = _two_medians(pk, tk, k)
    shift_p = _key_to_f32(med_p)
    shift_t = _key_to_f32(med_t)
    scale_p = jnp.sum(jnp.abs(p - shift_p) * maskf) / cnt_f
    pn = (p - shift_p) / scale_p
    scale_t = jnp.sum(jnp.abs(t - shift_t) * maskf) / cnt_f
    tn = (t - shift_t) / scale_t

    mask2 = tn > 0.0
    cnt2_f = jnp.sum(mask2.astype(jnp.float32))
    cnt2 = cnt2_f.astype(jnp.int32)
    res = jnp.where(mask2, jnp.abs(pn - tn), 0.0)
    max_index = jnp.minimum((_N - cnt2) + _TRIM_OFF, _N - 1)
    gi = max_index // _W
    gj = max_index - gi * _W
    ri = jax.lax.broadcasted_iota(jnp.int32, (_H, _W), 0)
    ci = jax.lax.broadcasted_iota(jnp.int32, (_H, _W), 1)
    max_res = jnp.sum(jnp.where((ri == gi) & (ci == gj), res, 0.0))
    res = jnp.where(res > max_res, 0.0, res)
    mae = jnp.where(cnt2 > 0, jnp.sum(res) / (2.0 * cnt2_f), 0.0)

    for g in range(4):
        pn_ref[g] = pn[:, 128 * g:128 * (g + 1)]
        tn_ref[g] = tn[:, 128 * g:128 * (g + 1)]
    g1 = _grad_term(pn_ref, tn_ref, 1)
    g2 = _grad_term(pn_ref, tn_ref, 2)
    g4 = _grad_term(pn_ref, tn_ref, 4)
    g8 = _grad_term(pn_ref, tn_ref, 8)

    lane = jax.lax.broadcasted_iota(jnp.int32, (1, 128), 1)
    vec = (
        jnp.where(lane == 0, mae, 0.0)
        + jnp.where(lane == 1, g1, 0.0)
        + jnp.where(lane == 2, g2, 0.0)
        + jnp.where(lane == 3, g4, 0.0)
        + jnp.where(lane == 4, g8, 0.0)
    )
    out_ref[0] = vec


@jax.jit
def kernel(prediction, target):
    b = prediction.shape[0]
    out = pl.pallas_call(
        _loss_kernel,
        grid=(b,),
        in_specs=[
            pl.BlockSpec((1, _H, _W), lambda i: (i, 0, 0)),
            pl.BlockSpec((1, _H, _W), lambda i: (i, 0, 0)),
        ],
        out_specs=pl.BlockSpec((1, 1, 128), lambda i: (i, 0, 0)),
        out_shape=jax.ShapeDtypeStruct((b, 1, 128), jnp.float32),
        scratch_shapes=[
            pltpu.VMEM((4, _H, 128), jnp.float32),
            pltpu.VMEM((4, _H, 128), jnp.float32),
        ],
    )(prediction, target)
    per_row = out[:, 0, :]
    loss = jnp.mean(per_row[:, 0]) + _ALPHA * (
        jnp.mean(per_row[:, 1])
        + jnp.mean(per_row[:, 2])
        + jnp.mean(per_row[:, 3])
        + jnp.mean(per_row[:, 4])
    )
    return loss


# speedup vs baseline: 41.2400x; 1.0067x over previous
import functools

import jax
import jax.numpy as jnp
from jax.experimental import pallas as pl
from jax.experimental.pallas import tpu as pltpu

_TRIM = 0.2
_ALPHA = 0.5
_SCALES = 4
_H = 512
_W = 512
_N = _H * _W
_TRIM_OFF = int((1 - _TRIM) * _N)


def _ukey(x):
    b = jax.lax.bitcast_convert_type(x, jnp.uint32)
    return jnp.where(b >= jnp.uint32(0x80000000), ~b, b ^ jnp.uint32(0x80000000))


def _key_to_f32(r):
    b = jnp.where(r >= jnp.uint32(0x80000000), r ^ jnp.uint32(0x80000000), ~r)
    return jax.lax.bitcast_convert_type(b, jnp.float32)


def _count16(plane, c):
    cb = (c ^ 0x8000).astype(jnp.int16)
    x = (plane < cb).astype(jnp.int16)
    rows = x.shape[0]
    while rows > 8:
        half = rows // 2
        x = x[:half] + x[half:]
        rows = half
    return jnp.sum(x.astype(jnp.int32))


def _bisect16(pa, pb, ka, kb):
    def body(i, carry):
        ra, rb, bla, blb = carry
        bit = jnp.int32(1) << (jnp.int32(15) - i)
        ca = ra | bit
        cb = rb | bit
        na = _count16(pa, ca)
        nb = _count16(pb, cb)
        ok_a = na <= ka
        ok_b = nb <= kb
        return (jnp.where(ok_a, ca, ra), jnp.where(ok_b, cb, rb),
                jnp.where(ok_a, na, bla), jnp.where(ok_b, nb, blb))

    init = (jnp.int32(0), jnp.int32(0), jnp.int32(0), jnp.int32(0))
    return jax.lax.fori_loop(0, 16, body, init)


def _biased16(u):
    return (u ^ jnp.uint32(0x8000)).astype(jnp.int16)


def _two_medians(keys_a, keys_b, k):
    hi_a = _biased16(keys_a >> 16)
    hi_b = _biased16(keys_b >> 16)
    rha, rhb, below_a, below_b = _bisect16(hi_a, hi_b, k, k)
    sent = jnp.int16(0x7FFF)
    lo_a = jnp.where(hi_a == (rha ^ 0x8000).astype(jnp.int16),
                     _biased16(keys_a & jnp.uint32(0xFFFF)), sent)
    lo_b = jnp.where(hi_b == (rhb ^ 0x8000).astype(jnp.int16),
                     _biased16(keys_b & jnp.uint32(0xFFFF)), sent)
    rla, rlb, _, _ = _bisect16(lo_a, lo_b, k - below_a, k - below_b)
    key_a = (rha.astype(jnp.uint32) << 16) | rla.astype(jnp.uint32)
    key_b = (rhb.astype(jnp.uint32) << 16) | rlb.astype(jnp.uint32)
    return key_a, key_b


def _grad_term(pn_ref, tn_ref, s):
    if s > 1:
        sl = pl.Slice(0, _H // s, s)
        ps = jnp.concatenate([pn_ref[g, sl, :] for g in range(4)], axis=1)
        ts = jnp.concatenate([tn_ref[g, sl, :] for g in range(4)], axis=1)
    else:
        ps, ts = pn_ref, tn_ref
    mg = (ts > 0.0).astype(jnp.float32)
    if s > 1:
        ci = jax.lax.broadcasted_iota(jnp.int32, mg.shape, 1)
        csel = (ci & (s - 1)) == 0
        mg = jnp.where(csel, mg, 0.0)
        dx = jnp.abs(ps[:, s:] - ps[:, :-s])
        lx = jnp.sum(dx * (mg[:, s:] * mg[:, :-s]))
        dy = jnp.abs(ps[1:, :] - ps[:-1, :])
        ly = jnp.sum(dy * (mg[1:, :] * mg[:-1, :]))
    else:
        dx = jnp.abs(ps[:, 1:] - ps[:, :-1])
        lx = jnp.sum(dx * (mg[:, 1:] * mg[:, :-1]))
        dy = jnp.abs(ps[1:, :] - ps[:-1, :])
        ly = jnp.sum(dy * (mg[1:, :] * mg[:-1, :]))
    norm = jnp.sum(mg)
    return jnp.where(norm > 0.0, (lx + ly) / norm, 0.0)


def _loss_kernel(pred_ref, targ_ref, out_ref, pn_ref, tn_ref):
    p = pred_ref[0]
    t = targ_ref[0]
    mask = t > 0.0
    maskf = mask.astype(jnp.float32)
    cnt_f = jnp.sum(maskf)
    cnt = cnt_f.astype(jnp.int32)
    k = jnp.maximum((cnt - 1) // 2, 0)

    inf = jnp.float32(jnp.inf)

    pk = _ukey(jnp.where(mask, p, inf))
    tk = _ukey(jnp.where(mask, t, inf))
    med_p, med_t = _two_medians(pk, tk, k)
    shift_p = _key_to_f32(med_p)
    shift_t = _key_to_f32(med_t)
    scale_p = jnp.sum(jnp.abs(p - shift_p) * maskf) / cnt_f
    pn = (p - shift_p) / scale_p
    scale_t = jnp.sum(jnp.abs(t - shift_t) * maskf) / cnt_f
    tn = (t - shift_t) / scale_t

    mask2 = tn > 0.0
    cnt2_f = jnp.sum(mask2.astype(jnp.float32))
    cnt2 = cnt2_f.astype(jnp.int32)
    res = jnp.where(mask2, jnp.abs(pn - tn), 0.0)
    max_index = jnp.minimum((_N - cnt2) + _TRIM_OFF, _N - 1)
    gi = max_index // _W
    gj = max_index - gi * _W
    ri = jax.lax.broadcasted_iota(jnp.int32, (_H, _W), 0)
    ci = jax.lax.broadcasted_iota(jnp.int32, (_H, _W), 1)
    max_res = jnp.sum(jnp.where((ri == gi) & (ci == gj), res, 0.0))
    res = jnp.where(res > max_res, 0.0, res)
    mae = jnp.where(cnt2 > 0, jnp.sum(res) / (2.0 * cnt2_f), 0.0)

    for g in range(4):
        pn_ref[g] = pn[:, 128 * g:128 * (g + 1)]
        tn_ref[g] = tn[:, 128 * g:128 * (g + 1)]
    g1 = _grad_term(pn, tn, 1)
    g2 = _grad_term(pn_ref, tn_ref, 2)
    g4 = _grad_term(pn_ref, tn_ref, 4)
    g8 = _grad_term(pn_ref, tn_ref, 8)

    lane = jax.lax.broadcasted_iota(jnp.int32, (1, 128), 1)
    vec = (
        jnp.where(lane == 0, mae, 0.0)
        + jnp.where(lane == 1, g1, 0.0)
        + jnp.where(lane == 2, g2, 0.0)
        + jnp.where(lane == 3, g4, 0.0)
        + jnp.where(lane == 4, g8, 0.0)
    )
    out_ref[0] = vec


@jax.jit
def kernel(prediction, target):
    b = prediction.shape[0]
    out = pl.pallas_call(
        _loss_kernel,
        grid=(b,),
        in_specs=[
            pl.BlockSpec((1, _H, _W), lambda i: (i, 0, 0)),
            pl.BlockSpec((1, _H, _W), lambda i: (i, 0, 0)),
        ],
        out_specs=pl.BlockSpec((1, 1, 128), lambda i: (i, 0, 0)),
        out_shape=jax.ShapeDtypeStruct((b, 1, 128), jnp.float32),
        scratch_shapes=[
            pltpu.VMEM((4, _H, 128), jnp.float32),
            pltpu.VMEM((4, _H, 128), jnp.float32),
        ],
    )(prediction, target)
    per_row = out[:, 0, :]
    loss = jnp.mean(per_row[:, 0]) + _ALPHA * (
        jnp.mean(per_row[:, 1])
        + jnp.mean(per_row[:, 2])
        + jnp.mean(per_row[:, 3])
        + jnp.mean(per_row[:, 4])
    )
    return loss
